# baseline (speedup 1.0000x reference)
import functools

import jax
import jax.numpy as jnp
from jax import lax
from jax.experimental import pallas as pl
from jax.experimental.pallas import tpu as pltpu
from jax.experimental.pallas import tpu_sc as plsc

N = 10000
D = 128
E = 320000
NC = 2
NS = 16
K = 128
SUP = 8
SG = 20
NSUPT = SG + 1
LPT = NSUPT * SUP * K
EPT_RAW = E // NS
NPAD = 10240
RPT = NPAD // NS


def _sc_body(compute_counts, *refs):
    if compute_counts:
        (tab0, tab1, pidx0, pidx1,
         out0, out1, cnt0, cnt1,
         idxa, idxb, rowsa, rowsb, danum, ones, cstage, acc, cacc,
         gsem, ssem, csem, isem) = refs
    else:
        (tab0, tab1, pidx0, pidx1,
         out0, out1,
         idxa, idxb, rowsa, rowsb, danum, acc,
         gsem, ssem, csem, isem) = refs

    c = lax.axis_index("c")
    s = lax.axis_index("s")
    zero16 = jnp.zeros((16,), jnp.float32)
    one16 = jnp.ones((16,), jnp.float32)
    n16 = jnp.full((16,), N, jnp.int32)

    def _zrow(r, carry):
        for j in range(D // 16):
            rowsa[r, pl.ds(j * 16, 16)] = zero16
        return carry
    lax.fori_loop(0, K, _zrow, 0)
    for j in range(K // 16):
        danum[pl.ds(j * 16, 16)] = n16
    if compute_counts:
        for j in range(K // 16):
            ones[pl.ds(j * 16, 16)] = one16
        for j in range(RPT // 16):
            cstage[pl.ds(j * 16, 16)] = zero16

    base = s * RPT
    for q in range(RPT // K):
        pltpu.sync_copy(rowsa, acc.at[pl.ds(base + q * K, K)])
    if compute_counts:
        pltpu.sync_copy(cstage, cacc.at[pl.ds(base, RPT)])
    plsc.subcore_barrier()

    def _process(tab, pidx):
        def fire_idx(ib, sg):
            pltpu.async_copy(
                pidx.at[pl.ds((s * NSUPT + sg) * 2 * SUP, 2 * SUP)], ib, isem)

        def wait_idx(ib, sg):
            pltpu.make_async_copy(
                pidx.at[pl.ds((s * NSUPT + sg) * 2 * SUP, 2 * SUP)], ib,
                isem).wait()

        def fire_gather(ib, row, rb):
            pltpu.async_copy(tab.at[ib.at[row]], rb, gsem)

        def drain_gather(rb):
            pltpu.make_async_copy(tab.at[idxa.at[0]], rb, gsem).wait()

        def fire_scatter(ib, row, rb):
            pltpu.async_copy(rb, acc.at[ib.at[SUP + row]], ssem, add=True)
            if compute_counts:
                pltpu.async_copy(ones, cacc.at[ib.at[SUP + row]], csem, add=True)

        def drain_scatter(rb):
            pltpu.make_async_copy(rb, acc.at[danum], ssem).wait()
            if compute_counts:
                pltpu.make_async_copy(ones, cacc.at[danum], csem).wait()

        def step(ib_cur, ib_nxt, sg, j):
            rb_cur, rb_oth = (rowsa, rowsb) if j % 2 == 0 else (rowsb, rowsa)
            drain_gather(rb_cur)
            drain_scatter(rb_oth)
            if j == SUP - 1:
                wait_idx(ib_nxt, sg + 1)
                fire_gather(ib_nxt, 0, rb_oth)
            else:
                fire_gather(ib_cur, j + 1, rb_oth)
            fire_scatter(ib_cur, j, rb_cur)

        fire_idx(idxa, 0)
        wait_idx(idxa, 0)
        fire_gather(idxa, 0, rowsa)
        pltpu.async_copy(rowsb, acc.at[danum], ssem, add=True)
        if compute_counts:
            pltpu.async_copy(ones, cacc.at[danum], csem, add=True)

        def _pair(t, carry):
            sg0 = 2 * t
            step(idxa, idxb, sg0, 0)
            fire_idx(idxb, sg0 + 1)
            for j in range(1, SUP):
                step(idxa, idxb, sg0, j)
            step(idxb, idxa, sg0 + 1, 0)
            fire_idx(idxa, sg0 + 2)
            for j in range(1, SUP):
                step(idxb, idxa, sg0 + 1, j)
            return carry
        lax.fori_loop(0, SG // 2, _pair, 0)
        drain_gather(rowsa)
        drain_scatter(rowsb)
        fire_scatter(idxa, 0, rowsa)
        drain_scatter(rowsa)

    @pl.when(c == 0)
    def _():
        _process(tab0, pidx0)

    @pl.when(c == 1)
    def _():
        _process(tab1, pidx1)

    plsc.subcore_barrier()

    def _writeout(out, cnt_out):
        for q in range(RPT // K):
            r0 = base + q * K
            pltpu.sync_copy(acc.at[pl.ds(r0, K)], rowsa)
            pltpu.sync_copy(rowsa, out.at[pl.ds(r0, K)])
        if compute_counts:
            pltpu.sync_copy(cacc.at[pl.ds(base, RPT)], cstage)
            pltpu.sync_copy(cstage, cnt_out.at[pl.ds(base, RPT)])

    @pl.when(c == 0)
    def _():
        _writeout(out0, cnt0 if compute_counts else None)

    @pl.when(c == 1)
    def _():
        _writeout(out1, cnt1 if compute_counts else None)


def _make_sc_agg(compute_counts):
    out_type = [jax.ShapeDtypeStruct((NPAD, D), jnp.float32)] * 2
    if compute_counts:
        out_type += [jax.ShapeDtypeStruct((NPAD,), jnp.float32)] * 2
    scratch = [
        pltpu.VMEM((2 * SUP, K), jnp.int32),
        pltpu.VMEM((2 * SUP, K), jnp.int32),
        pltpu.VMEM((K, D), jnp.float32),
        pltpu.VMEM((K, D), jnp.float32),
        pltpu.VMEM((K,), jnp.int32),
    ]
    if compute_counts:
        scratch += [
            pltpu.VMEM((K,), jnp.float32),
            pltpu.VMEM((RPT,), jnp.float32),
        ]
    scratch += [pltpu.VMEM_SHARED((NPAD, D), jnp.float32)]
    if compute_counts:
        scratch += [pltpu.VMEM_SHARED((NPAD,), jnp.float32)]
    scratch += [pltpu.SemaphoreType.DMA] * 4
    mesh = plsc.VectorSubcoreMesh(
        core_axis_name="c", subcore_axis_name="s", num_cores=NC, num_subcores=NS)
    return pl.kernel(
        functools.partial(_sc_body, compute_counts),
        out_type=tuple(out_type),
        mesh=mesh,
        scratch_types=tuple(scratch),
    )


_sc_agg_counts = _make_sc_agg(True)
_sc_agg = _make_sc_agg(False)


def _tc_sage_body(relu, agg_ref, cnt_ref, x_ref, wl_ref, wr_ref, b_ref, out_ref):
    inv = 1.0 / jnp.maximum(cnt_ref[...], 1.0)
    mean = agg_ref[...] * inv
    dn = (((1,), (1,)), ((), ()))
    out = (lax.dot_general(mean, wl_ref[...], dn, preferred_element_type=jnp.float32)
           + lax.dot_general(x_ref[...], wr_ref[...], dn, preferred_element_type=jnp.float32)
           + b_ref[...])
    if relu:
        out = jnp.maximum(out, 0.0)
    out_ref[...] = out


def _tc_sage(agg, cnt, x, wl, wr, b, relu):
    bt = 2000
    return pl.pallas_call(
        functools.partial(_tc_sage_body, relu),
        grid=(N // bt,),
        in_specs=[
            pl.BlockSpec((bt, D), lambda i: (i, 0)),
            pl.BlockSpec((bt, 1), lambda i: (i, 0)),
            pl.BlockSpec((bt, D), lambda i: (i, 0)),
            pl.BlockSpec((D, D), lambda i: (0, 0)),
            pl.BlockSpec((D, D), lambda i: (0, 0)),
            pl.BlockSpec((1, D), lambda i: (0, 0)),
        ],
        out_specs=pl.BlockSpec((bt, D), lambda i: (i, 0)),
        out_shape=jax.ShapeDtypeStruct((N, D), jnp.float32),
    )(agg, cnt, x, wl, wr, b)


def _pack_idx(src, dst):
    s2 = jnp.pad(src.astype(jnp.int32).reshape(NS, EPT_RAW),
                 ((0, 0), (0, LPT - EPT_RAW))).reshape(NS, NSUPT, SUP, K)
    d2 = jnp.pad(dst.astype(jnp.int32).reshape(NS, EPT_RAW),
                 ((0, 0), (0, LPT - EPT_RAW)),
                 constant_values=N).reshape(NS, NSUPT, SUP, K)
    return jnp.concatenate([s2, d2], axis=2).reshape(NS * NSUPT * 2 * SUP, K)


def kernel(x_author, x_paper, edge_index_writes, edge_index_written_by,
           W1_wp_l, W1_wp_r, b1_wp, W1_pa_l, W1_pa_r, b1_pa,
           W2_wp_l, W2_wp_r, b2_wp, W2_pa_l, W2_pa_r, b2_pa):
    pw = _pack_idx(edge_index_writes[0], edge_index_writes[1])
    pb = _pack_idx(edge_index_written_by[0], edge_index_written_by[1])

    aggw, aggb, cntw, cntb = _sc_agg_counts(x_author, x_paper, pw, pb)
    cw = cntw[:N, None]
    cb = cntb[:N, None]

    p1 = _tc_sage(aggw[:N], cw, x_paper, W1_wp_l, W1_wp_r, b1_wp[None, :], True)
    a1 = _tc_sage(aggb[:N], cb, x_author, W1_pa_l, W1_pa_r, b1_pa[None, :], True)

    agg2w, agg2b = _sc_agg(a1, p1, pw, pb)

    p2 = _tc_sage(agg2w[:N], cw, p1, W2_wp_l, W2_wp_r, b2_wp[None, :], False)
    a2 = _tc_sage(agg2b[:N], cb, a1, W2_pa_l, W2_pa_r, b2_pa[None, :], False)
    return (a2, p2)


# speedup vs baseline: 1.0019x; 1.0019x over previous
import functools

import jax
import jax.numpy as jnp
from jax import lax
from jax.experimental import pallas as pl
from jax.experimental.pallas import tpu as pltpu
from jax.experimental.pallas import tpu_sc as plsc

N = 10000
D = 128
E = 320000
NC = 2
NS = 16
K = 128
SUP = 8
SG = 20
NSUPT = SG + 1
LPT = NSUPT * SUP * K
EPT_RAW = E // NS
NPAD = 10240
RPT = NPAD // NS


def _sc_body(compute_counts, *refs):
    if compute_counts:
        (tab0, tab1, pidx0, pidx1,
         out0, out1, cnt0, cnt1,
         idxs, rows, danum, ones, cstage, acc, cacc,
         gsem, ssem, csem, isem) = refs
    else:
        (tab0, tab1, pidx0, pidx1,
         out0, out1,
         idxs, rows, danum, acc,
         gsem, ssem, csem, isem) = refs

    c = lax.axis_index("c")
    s = lax.axis_index("s")
    zero16 = jnp.zeros((16,), jnp.float32)
    one16 = jnp.ones((16,), jnp.float32)
    n16 = jnp.full((16,), N, jnp.int32)

    def _zrow(r, carry):
        for j in range(D // 16):
            rows[0, r, pl.ds(j * 16, 16)] = zero16
        return carry
    lax.fori_loop(0, K, _zrow, 0)
    for j in range(K // 16):
        danum[pl.ds(j * 16, 16)] = n16
    if compute_counts:
        for j in range(K // 16):
            ones[pl.ds(j * 16, 16)] = one16
        for j in range(RPT // 16):
            cstage[pl.ds(j * 16, 16)] = zero16

    base = s * RPT
    for q in range(RPT // K):
        pltpu.sync_copy(rows.at[0], acc.at[pl.ds(base + q * K, K)])
    if compute_counts:
        pltpu.sync_copy(cstage, cacc.at[pl.ds(base, RPT)])
    plsc.subcore_barrier()

    def _process(tab, pidx):
        def fire_idx(sg):
            pltpu.async_copy(
                pidx.at[pl.ds((s * NSUPT + sg) * 2 * SUP, 2 * SUP)],
                idxs.at[lax.rem(sg, 2)], isem)

        def wait_idx(sg):
            pltpu.make_async_copy(
                pidx.at[pl.ds((s * NSUPT + sg) * 2 * SUP, 2 * SUP)],
                idxs.at[lax.rem(sg, 2)], isem).wait()

        def fire_gather(i):
            ib = idxs.at[lax.rem(lax.div(i, SUP), 2)]
            pltpu.async_copy(tab.at[ib.at[lax.rem(i, SUP)]],
                             rows.at[lax.rem(i, 2)], gsem)

        def drain_gather(i):
            pltpu.make_async_copy(tab.at[danum],
                                  rows.at[lax.rem(i, 2)], gsem).wait()

        def fire_scatter(i):
            ib = idxs.at[lax.rem(lax.div(i, SUP), 2)]
            rb = rows.at[lax.rem(i, 2)]
            pltpu.async_copy(rb, acc.at[ib.at[SUP + lax.rem(i, SUP)]],
                             ssem, add=True)
            if compute_counts:
                pltpu.async_copy(ones, cacc.at[ib.at[SUP + lax.rem(i, SUP)]],
                                 csem, add=True)

        def drain_scatter(i):
            pltpu.make_async_copy(rows.at[lax.rem(i, 2)], acc.at[danum],
                                  ssem).wait()
            if compute_counts:
                pltpu.make_async_copy(ones, cacc.at[danum], csem).wait()

        fire_idx(0)
        wait_idx(0)
        fire_gather(0)
        pltpu.async_copy(rows.at[1], acc.at[danum], ssem, add=True)
        if compute_counts:
            pltpu.async_copy(ones, cacc.at[danum], csem, add=True)

        def _step(i, carry):
            drain_gather(i)
            drain_scatter(i - 1)

            @pl.when(lax.rem(i, SUP) == 0)
            def _():
                fire_idx(lax.div(i, SUP) + 1)

            @pl.when(lax.rem(i, SUP) == SUP - 1)
            def _():
                wait_idx(lax.div(i, SUP) + 1)
            fire_gather(i + 1)
            fire_scatter(i)
            return carry
        lax.fori_loop(0, SG * SUP, _step, 0)
        i_last = SG * SUP
        drain_gather(i_last)
        drain_scatter(i_last - 1)
        fire_scatter(i_last)
        drain_scatter(i_last)

    @pl.when(c == 0)
    def _():
        _process(tab0, pidx0)

    @pl.when(c == 1)
    def _():
        _process(tab1, pidx1)

    plsc.subcore_barrier()

    def _writeout(out, cnt_out):
        for q in range(RPT // K):
            r0 = base + q * K
            pltpu.sync_copy(acc.at[pl.ds(r0, K)], rows.at[0])
            pltpu.sync_copy(rows.at[0], out.at[pl.ds(r0, K)])
        if compute_counts:
            pltpu.sync_copy(cacc.at[pl.ds(base, RPT)], cstage)
            pltpu.sync_copy(cstage, cnt_out.at[pl.ds(base, RPT)])

    @pl.when(c == 0)
    def _():
        _writeout(out0, cnt0 if compute_counts else None)

    @pl.when(c == 1)
    def _():
        _writeout(out1, cnt1 if compute_counts else None)


def _make_sc_agg(compute_counts):
    out_type = [jax.ShapeDtypeStruct((NPAD, D), jnp.float32)] * 2
    if compute_counts:
        out_type += [jax.ShapeDtypeStruct((NPAD,), jnp.float32)] * 2
    scratch = [
        pltpu.VMEM((2, 2 * SUP, K), jnp.int32),
        pltpu.VMEM((2, K, D), jnp.float32),
        pltpu.VMEM((K,), jnp.int32),
    ]
    if compute_counts:
        scratch += [
            pltpu.VMEM((K,), jnp.float32),
            pltpu.VMEM((RPT,), jnp.float32),
        ]
    scratch += [pltpu.VMEM_SHARED((NPAD, D), jnp.float32)]
    if compute_counts:
        scratch += [pltpu.VMEM_SHARED((NPAD,), jnp.float32)]
    scratch += [pltpu.SemaphoreType.DMA] * 4
    mesh = plsc.VectorSubcoreMesh(
        core_axis_name="c", subcore_axis_name="s", num_cores=NC, num_subcores=NS)
    return pl.kernel(
        functools.partial(_sc_body, compute_counts),
        out_type=tuple(out_type),
        mesh=mesh,
        scratch_types=tuple(scratch),
    )


_sc_agg_counts = _make_sc_agg(True)
_sc_agg = _make_sc_agg(False)


def _tc_sage_body(relu, agg_ref, cnt_ref, x_ref, wl_ref, wr_ref, b_ref, out_ref):
    inv = 1.0 / jnp.maximum(cnt_ref[...], 1.0)
    mean = agg_ref[...] * inv
    dn = (((1,), (1,)), ((), ()))
    out = (lax.dot_general(mean, wl_ref[...], dn, preferred_element_type=jnp.float32)
           + lax.dot_general(x_ref[...], wr_ref[...], dn, preferred_element_type=jnp.float32)
           + b_ref[...])
    if relu:
        out = jnp.maximum(out, 0.0)
    out_ref[...] = out


def _tc_sage(agg, cnt, x, wl, wr, b, relu):
    bt = 2000
    return pl.pallas_call(
        functools.partial(_tc_sage_body, relu),
        grid=(N // bt,),
        in_specs=[
            pl.BlockSpec((bt, D), lambda i: (i, 0)),
            pl.BlockSpec((bt, 1), lambda i: (i, 0)),
            pl.BlockSpec((bt, D), lambda i: (i, 0)),
            pl.BlockSpec((D, D), lambda i: (0, 0)),
            pl.BlockSpec((D, D), lambda i: (0, 0)),
            pl.BlockSpec((1, D), lambda i: (0, 0)),
        ],
        out_specs=pl.BlockSpec((bt, D), lambda i: (i, 0)),
        out_shape=jax.ShapeDtypeStruct((N, D), jnp.float32),
    )(agg, cnt, x, wl, wr, b)


def _pack_idx(src, dst):
    s2 = jnp.pad(src.astype(jnp.int32).reshape(NS, EPT_RAW),
                 ((0, 0), (0, LPT - EPT_RAW))).reshape(NS, NSUPT, SUP, K)
    d2 = jnp.pad(dst.astype(jnp.int32).reshape(NS, EPT_RAW),
                 ((0, 0), (0, LPT - EPT_RAW)),
                 constant_values=N).reshape(NS, NSUPT, SUP, K)
    return jnp.concatenate([s2, d2], axis=2).reshape(NS * NSUPT * 2 * SUP, K)


def kernel(x_author, x_paper, edge_index_writes, edge_index_written_by,
           W1_wp_l, W1_wp_r, b1_wp, W1_pa_l, W1_pa_r, b1_pa,
           W2_wp_l, W2_wp_r, b2_wp, W2_pa_l, W2_pa_r, b2_pa):
    pw = _pack_idx(edge_index_writes[0], edge_index_writes[1])
    pb = _pack_idx(edge_index_written_by[0], edge_index_written_by[1])

    aggw, aggb, cntw, cntb = _sc_agg_counts(x_author, x_paper, pw, pb)
    cw = cntw[:N, None]
    cb = cntb[:N, None]

    p1 = _tc_sage(aggw[:N], cw, x_paper, W1_wp_l, W1_wp_r, b1_wp[None, :], True)
    a1 = _tc_sage(aggb[:N], cb, x_author, W1_pa_l, W1_pa_r, b1_pa[None, :], True)

    agg2w, agg2b = _sc_agg(a1, p1, pw, pb)

    p2 = _tc_sage(agg2w[:N], cw, p1, W2_wp_l, W2_wp_r, b2_wp[None, :], False)
    a2 = _tc_sage(agg2b[:N], cb, a1, W2_pa_l, W2_pa_r, b2_pa[None, :], False)
    return (a2, p2)


# speedup vs baseline: 1.0058x; 1.0039x over previous
import functools

import jax
import jax.numpy as jnp
from jax import lax
from jax.experimental import pallas as pl
from jax.experimental.pallas import tpu as pltpu
from jax.experimental.pallas import tpu_sc as plsc

N = 10000
D = 128
E = 320000
NC = 2
NS = 16
K = 128
NCH = 160
CPTOT = NCH + 1
LPT = CPTOT * K
EPT_RAW = E // NS
NPAD = 10240
RPT = NPAD // NS


def _sc_body(compute_counts, *refs):
    if compute_counts:
        (tab0, tab1, src0, dst0, src1, dst1,
         out0, out1, cnt0, cnt1,
         isa, ida, isb, idb, rows0, rows1, danum, ones, cstage, acc, cacc,
         gsem, ssem, csem, isem) = refs
    else:
        (tab0, tab1, src0, dst0, src1, dst1,
         out0, out1,
         isa, ida, isb, idb, rows0, rows1, danum, acc,
         gsem, ssem, csem, isem) = refs

    c = lax.axis_index("c")
    s = lax.axis_index("s")
    zero16 = jnp.zeros((16,), jnp.float32)
    one16 = jnp.ones((16,), jnp.float32)
    n16 = jnp.full((16,), N, jnp.int32)

    def _zrow(r, carry):
        for j in range(D // 16):
            rows0[r, pl.ds(j * 16, 16)] = zero16
        return carry
    lax.fori_loop(0, K, _zrow, 0)
    for j in range(K // 16):
        danum[pl.ds(j * 16, 16)] = n16
    if compute_counts:
        for j in range(K // 16):
            ones[pl.ds(j * 16, 16)] = one16
        for j in range(RPT // 16):
            cstage[pl.ds(j * 16, 16)] = zero16

    base = s * RPT
    for q in range(RPT // K):
        pltpu.sync_copy(rows0, acc.at[pl.ds(base + q * K, K)])
    if compute_counts:
        pltpu.sync_copy(cstage, cacc.at[pl.ds(base, RPT)])
    plsc.subcore_barrier()

    def _process(tab, srcarr, dstarr):
        def fire_idx(i, isref, idref):
            off = s * LPT + i * K
            pltpu.async_copy(srcarr.at[pl.ds(off, K)], isref, isem)
            pltpu.async_copy(dstarr.at[pl.ds(off, K)], idref, isem)

        def wait_idx(isref, idref):
            pltpu.make_async_copy(srcarr.at[pl.ds(0, K)], isref, isem).wait()
            pltpu.make_async_copy(dstarr.at[pl.ds(0, K)], idref, isem).wait()

        def fire_gather(isref, rb):
            pltpu.async_copy(tab.at[isref], rb, gsem)

        def drain_gather(rb):
            pltpu.make_async_copy(tab.at[danum], rb, gsem).wait()

        def fire_scatter(idref, rb):
            pltpu.async_copy(rb, acc.at[idref], ssem, add=True)
            if compute_counts:
                pltpu.async_copy(ones, cacc.at[idref], csem, add=True)

        def drain_scatter(rb):
            pltpu.make_async_copy(rb, acc.at[danum], ssem).wait()
            if compute_counts:
                pltpu.make_async_copy(ones, cacc.at[danum], csem).wait()

        fire_idx(0, isa, ida)
        wait_idx(isa, ida)
        fire_gather(isa, rows0)
        pltpu.async_copy(rows1, acc.at[danum], ssem, add=True)
        if compute_counts:
            pltpu.async_copy(ones, cacc.at[danum], csem, add=True)

        def halfstep(i, cur, nxt):
            (is_c, id_c, rb_c), (is_n, id_n, rb_n) = cur, nxt
            drain_scatter(rb_n)
            fire_idx(i + 1, is_n, id_n)
            drain_gather(rb_c)
            fire_scatter(id_c, rb_c)
            wait_idx(is_n, id_n)
            fire_gather(is_n, rb_n)

        seta = (isa, ida, rows0)
        setb = (isb, idb, rows1)

        def _pairstep(t, carry):
            halfstep(2 * t, seta, setb)
            halfstep(2 * t + 1, setb, seta)
            return carry
        lax.fori_loop(0, NCH // 2, _pairstep, 0)
        drain_scatter(rows1)
        drain_gather(rows0)
        fire_scatter(ida, rows0)
        drain_scatter(rows0)

    @pl.when(c == 0)
    def _():
        _process(tab0, src0, dst0)

    @pl.when(c == 1)
    def _():
        _process(tab1, src1, dst1)

    plsc.subcore_barrier()

    def _writeout(out, cnt_out):
        for q in range(RPT // K):
            r0 = base + q * K
            pltpu.sync_copy(acc.at[pl.ds(r0, K)], rows0)
            pltpu.sync_copy(rows0, out.at[pl.ds(r0, K)])
        if compute_counts:
            pltpu.sync_copy(cacc.at[pl.ds(base, RPT)], cstage)
            pltpu.sync_copy(cstage, cnt_out.at[pl.ds(base, RPT)])

    @pl.when(c == 0)
    def _():
        _writeout(out0, cnt0 if compute_counts else None)

    @pl.when(c == 1)
    def _():
        _writeout(out1, cnt1 if compute_counts else None)


def _make_sc_agg(compute_counts):
    out_type = [jax.ShapeDtypeStruct((NPAD, D), jnp.float32)] * 2
    if compute_counts:
        out_type += [jax.ShapeDtypeStruct((NPAD,), jnp.float32)] * 2
    scratch = [
        pltpu.VMEM((K,), jnp.int32),
        pltpu.VMEM((K,), jnp.int32),
        pltpu.VMEM((K,), jnp.int32),
        pltpu.VMEM((K,), jnp.int32),
        pltpu.VMEM((K, D), jnp.float32),
        pltpu.VMEM((K, D), jnp.float32),
        pltpu.VMEM((K,), jnp.int32),
    ]
    if compute_counts:
        scratch += [
            pltpu.VMEM((K,), jnp.float32),
            pltpu.VMEM((RPT,), jnp.float32),
        ]
    scratch += [pltpu.VMEM_SHARED((NPAD, D), jnp.float32)]
    if compute_counts:
        scratch += [pltpu.VMEM_SHARED((NPAD,), jnp.float32)]
    scratch += [pltpu.SemaphoreType.DMA] * 4
    mesh = plsc.VectorSubcoreMesh(
        core_axis_name="c", subcore_axis_name="s", num_cores=NC, num_subcores=NS)
    return pl.kernel(
        functools.partial(_sc_body, compute_counts),
        out_type=tuple(out_type),
        mesh=mesh,
        scratch_types=tuple(scratch),
    )


_sc_agg_counts = _make_sc_agg(True)
_sc_agg = _make_sc_agg(False)


def _tc_sage_body(relu, agg_ref, cnt_ref, x_ref, wl_ref, wr_ref, b_ref, out_ref):
    inv = 1.0 / jnp.maximum(cnt_ref[...], 1.0)
    mean = agg_ref[...] * inv
    dn = (((1,), (1,)), ((), ()))
    out = (lax.dot_general(mean, wl_ref[...], dn, preferred_element_type=jnp.float32)
           + lax.dot_general(x_ref[...], wr_ref[...], dn, preferred_element_type=jnp.float32)
           + b_ref[...])
    if relu:
        out = jnp.maximum(out, 0.0)
    out_ref[...] = out


def _tc_sage(agg, cnt, x, wl, wr, b, relu):
    bt = 2000
    return pl.pallas_call(
        functools.partial(_tc_sage_body, relu),
        grid=(N // bt,),
        in_specs=[
            pl.BlockSpec((bt, D), lambda i: (i, 0)),
            pl.BlockSpec((bt, 1), lambda i: (i, 0)),
            pl.BlockSpec((bt, D), lambda i: (i, 0)),
            pl.BlockSpec((D, D), lambda i: (0, 0)),
            pl.BlockSpec((D, D), lambda i: (0, 0)),
            pl.BlockSpec((1, D), lambda i: (0, 0)),
        ],
        out_specs=pl.BlockSpec((bt, D), lambda i: (i, 0)),
        out_shape=jax.ShapeDtypeStruct((N, D), jnp.float32),
    )(agg, cnt, x, wl, wr, b)


def _pad_edges(v, fill):
    v = v.astype(jnp.int32).reshape(NS, EPT_RAW)
    v = jnp.pad(v, ((0, 0), (0, LPT - EPT_RAW)), constant_values=fill)
    return v.reshape(NS * LPT)


def kernel(x_author, x_paper, edge_index_writes, edge_index_written_by,
           W1_wp_l, W1_wp_r, b1_wp, W1_pa_l, W1_pa_r, b1_pa,
           W2_wp_l, W2_wp_r, b2_wp, W2_pa_l, W2_pa_r, b2_pa):
    srcw = _pad_edges(edge_index_writes[0], 0)
    dstw = _pad_edges(edge_index_writes[1], N)
    srcb = _pad_edges(edge_index_written_by[0], 0)
    dstb = _pad_edges(edge_index_written_by[1], N)

    aggw, aggb, cntw, cntb = _sc_agg_counts(
        x_author, x_paper, srcw, dstw, srcb, dstb)
    cw = cntw[:N, None]
    cb = cntb[:N, None]

    p1 = _tc_sage(aggw[:N], cw, x_paper, W1_wp_l, W1_wp_r, b1_wp[None, :], True)
    a1 = _tc_sage(aggb[:N], cb, x_author, W1_pa_l, W1_pa_r, b1_pa[None, :], True)

    agg2w, agg2b = _sc_agg(a1, p1, srcw, dstw, srcb, dstb)

    p2 = _tc_sage(agg2w[:N], cw, p1, W2_wp_l, W2_wp_r, b2_wp[None, :], False)
    a2 = _tc_sage(agg2b[:N], cb, a1, W2_pa_l, W2_pa_r, b2_pa[None, :], False)
    return (a2, p2)
